# baseline (speedup 1.0000x reference)
import jax
import jax.numpy as jnp
from jax.experimental import pallas as pl

_N_SPHERES = 64
_TOTAL = 512
_ROWS = 512
_HI = jax.lax.Precision.HIGHEST


def _router_body(ns_ref, pos_ref, pc_ref, ct_ref, probs_ref, choice_ref):
    neg_s = ns_ref[...]

    dx = pos_ref[:, 0:1] - ct_ref[0:1, :]
    dy = pos_ref[:, 1:2] - ct_ref[1:2, :]
    dz = pos_ref[:, 2:3] - ct_ref[2:3, :]
    d_sq = (dx * dx + dy * dy) + dz * dz
    logits = d_sq / neg_s

    lpc = jnp.log(pc_ref[...] + 1e-10)
    logits = jnp.concatenate(
        [logits[:, g * _N_SPHERES:(g + 1) * _N_SPHERES] + lpc[:, g:g + 1]
         for g in range(8)], axis=1)

    lane = jax.lax.broadcasted_iota(jnp.int32, (1, _TOTAL), 1)
    m = jnp.max(logits, axis=-1, keepdims=True)
    cand = jnp.where(logits == m, lane, _TOTAL)
    choice = jnp.min(cand, axis=-1, keepdims=True)
    probs_ref[...] = (lane == choice).astype(jnp.float32)
    choice_ref[...] = choice.reshape(_ROWS)


def kernel(pos_3d, temperature, parent_choice, hard, centers, log_radii):
    del hard, log_radii
    b = pos_3d.shape[0]
    neg_s = (-(2.0 * temperature**2 + 1e-8)).reshape(1, 1).astype(jnp.float32)
    ct = centers.T
    grid = (b // _ROWS,)
    probs, choice = pl.pallas_call(
        _router_body,
        grid=grid,
        in_specs=[
            pl.BlockSpec((1, 1), lambda i: (0, 0)),
            pl.BlockSpec((_ROWS, 3), lambda i: (i, 0)),
            pl.BlockSpec((_ROWS, 8), lambda i: (i, 0)),
            pl.BlockSpec((3, _TOTAL), lambda i: (0, 0)),
        ],
        out_specs=[
            pl.BlockSpec((_ROWS, _TOTAL), lambda i: (i, 0)),
            pl.BlockSpec((_ROWS,), lambda i: (i,)),
        ],
        out_shape=[
            jax.ShapeDtypeStruct((b, _TOTAL), jnp.float32),
            jax.ShapeDtypeStruct((b,), jnp.int32),
        ],
    )(neg_s, pos_3d, parent_choice, ct)
    return probs, choice


# speedup vs baseline: 1.2035x; 1.2035x over previous
import jax
import jax.numpy as jnp
from jax.experimental import pallas as pl

_N_SPHERES = 64
_TOTAL = 512
_ROWS = 512


def _router_body(ns_ref, posT_ref, pcT_ref, ct_ref, probs_ref, choice_ref):
    neg_s = ns_ref[...]

    px = posT_ref[0:1, :].reshape(_ROWS, 1)
    py = posT_ref[1:2, :].reshape(_ROWS, 1)
    pz = posT_ref[2:3, :].reshape(_ROWS, 1)
    dx = px - ct_ref[0:1, :]
    dy = py - ct_ref[1:2, :]
    dz = pz - ct_ref[2:3, :]
    d_sq = (dx * dx + dy * dy) + dz * dz
    logits = d_sq / neg_s

    lpc = jnp.log(pcT_ref[...] + 1e-10)
    logits = jnp.concatenate(
        [logits[:, g * _N_SPHERES:(g + 1) * _N_SPHERES]
         + lpc[g:g + 1, :].reshape(_ROWS, 1)
         for g in range(8)], axis=1)

    lane = jax.lax.broadcasted_iota(jnp.int32, (1, _TOTAL), 1)
    m = jnp.max(logits, axis=-1, keepdims=True)
    cand = jnp.where(logits == m, lane, _TOTAL)
    choice = jnp.min(cand, axis=-1, keepdims=True)
    probs_ref[...] = (lane == choice).astype(jnp.float32)
    choice_ref[...] = choice.reshape(_ROWS)


def kernel(pos_3d, temperature, parent_choice, hard, centers, log_radii):
    del hard, log_radii
    b = pos_3d.shape[0]
    neg_s = (-(2.0 * temperature**2 + 1e-8)).reshape(1, 1).astype(jnp.float32)
    posT = pos_3d.T
    pcT = parent_choice.T
    ct = centers.T
    grid = (b // _ROWS,)
    probs, choice = pl.pallas_call(
        _router_body,
        grid=grid,
        in_specs=[
            pl.BlockSpec((1, 1), lambda i: (0, 0)),
            pl.BlockSpec((3, _ROWS), lambda i: (0, i)),
            pl.BlockSpec((8, _ROWS), lambda i: (0, i)),
            pl.BlockSpec((3, _TOTAL), lambda i: (0, 0)),
        ],
        out_specs=[
            pl.BlockSpec((_ROWS, _TOTAL), lambda i: (i, 0)),
            pl.BlockSpec((_ROWS,), lambda i: (i,)),
        ],
        out_shape=[
            jax.ShapeDtypeStruct((b, _TOTAL), jnp.float32),
            jax.ShapeDtypeStruct((b,), jnp.int32),
        ],
    )(neg_s, posT, pcT, ct)
    return probs, choice


# speedup vs baseline: 1.8511x; 1.5381x over previous
import jax
import jax.numpy as jnp
from jax.experimental import pallas as pl

_N_SPHERES = 64
_TOTAL = 512
_ROWS = 512


def _router_body(ns_ref, posT_ref, pcT_ref, c0_ref, probs_ref, choice_ref):
    neg_s = ns_ref[...]

    cx = c0_ref[:, 0:1]
    cy = c0_ref[:, 1:2]
    cz = c0_ref[:, 2:3]
    px = posT_ref[0:1, :]
    py = posT_ref[1:2, :]
    pz = posT_ref[2:3, :]
    dx = px - cx
    dy = py - cy
    dz = pz - cz
    d_sq = (dx * dx + dy * dy) + dz * dz
    v = d_sq / neg_s

    lpc = jnp.log(pcT_ref[...] + 1e-10)
    logits = jnp.concatenate(
        [v[g * _N_SPHERES:(g + 1) * _N_SPHERES, :] + lpc[g:g + 1, :]
         for g in range(8)], axis=0)

    srow = jax.lax.broadcasted_iota(jnp.int32, (_TOTAL, _ROWS), 0)
    m = jnp.max(logits, axis=0, keepdims=True)
    cand = jnp.where(logits == m, srow, _TOTAL)
    choiceT = jnp.min(cand, axis=0, keepdims=True)
    choice_ref[...] = choiceT.reshape(_ROWS)

    choice_col = choiceT.reshape(_ROWS, 1)
    lane = jax.lax.broadcasted_iota(jnp.int32, (_ROWS, _TOTAL), 1)
    probs_ref[...] = (lane == choice_col).astype(jnp.float32)


def kernel(pos_3d, temperature, parent_choice, hard, centers, log_radii):
    del hard, log_radii
    b = pos_3d.shape[0]
    neg_s = (-(2.0 * temperature**2 + 1e-8)).reshape(1, 1).astype(jnp.float32)
    posT = pos_3d.T
    pcT = parent_choice.T
    grid = (b // _ROWS,)
    probs, choice = pl.pallas_call(
        _router_body,
        grid=grid,
        in_specs=[
            pl.BlockSpec((1, 1), lambda i: (0, 0)),
            pl.BlockSpec((3, _ROWS), lambda i: (0, i)),
            pl.BlockSpec((8, _ROWS), lambda i: (0, i)),
            pl.BlockSpec((_TOTAL, 3), lambda i: (0, 0)),
        ],
        out_specs=[
            pl.BlockSpec((_ROWS, _TOTAL), lambda i: (i, 0)),
            pl.BlockSpec((_ROWS,), lambda i: (i,)),
        ],
        out_shape=[
            jax.ShapeDtypeStruct((b, _TOTAL), jnp.float32),
            jax.ShapeDtypeStruct((b,), jnp.int32),
        ],
    )(neg_s, posT, pcT, centers)
    return probs, choice


# speedup vs baseline: 2.2481x; 1.2144x over previous
import jax
import jax.numpy as jnp
from jax.experimental import pallas as pl

_N_SPHERES = 64
_TOTAL = 512
_ROWS = 1024


def _router_body(ns_ref, posT_ref, pcT_ref, c0_ref, probs_ref, choice_ref):
    neg_s = ns_ref[...]

    cx = c0_ref[:, 0:1]
    cy = c0_ref[:, 1:2]
    cz = c0_ref[:, 2:3]
    px = posT_ref[0:1, :]
    py = posT_ref[1:2, :]
    pz = posT_ref[2:3, :]
    dx = px - cx
    dy = py - cy
    dz = pz - cz
    d_sq = (dx * dx + dy * dy) + dz * dz
    v = d_sq / neg_s

    lpc = jnp.log(pcT_ref[...] + 1e-10)
    logits = jnp.concatenate(
        [v[g * _N_SPHERES:(g + 1) * _N_SPHERES, :] + lpc[g:g + 1, :]
         for g in range(8)], axis=0)

    srow = jax.lax.broadcasted_iota(jnp.int32, (_TOTAL, _ROWS), 0)
    m = jnp.max(logits, axis=0, keepdims=True)
    cand = jnp.where(logits == m, srow, _TOTAL)
    choiceT = jnp.min(cand, axis=0, keepdims=True)
    choice_ref[...] = choiceT.reshape(_ROWS)

    choice_col = choiceT.reshape(_ROWS, 1)
    lane = jax.lax.broadcasted_iota(jnp.int32, (_ROWS, _TOTAL), 1)
    probs_ref[...] = (lane == choice_col).astype(jnp.float32)


def kernel(pos_3d, temperature, parent_choice, hard, centers, log_radii):
    del hard, log_radii
    b = pos_3d.shape[0]
    neg_s = (-(2.0 * temperature**2 + 1e-8)).reshape(1, 1).astype(jnp.float32)
    posT = pos_3d.T
    pcT = parent_choice.T
    grid = (b // _ROWS,)
    probs, choice = pl.pallas_call(
        _router_body,
        grid=grid,
        in_specs=[
            pl.BlockSpec((1, 1), lambda i: (0, 0)),
            pl.BlockSpec((3, _ROWS), lambda i: (0, i)),
            pl.BlockSpec((8, _ROWS), lambda i: (0, i)),
            pl.BlockSpec((_TOTAL, 3), lambda i: (0, 0)),
        ],
        out_specs=[
            pl.BlockSpec((_ROWS, _TOTAL), lambda i: (i, 0)),
            pl.BlockSpec((_ROWS,), lambda i: (i,)),
        ],
        out_shape=[
            jax.ShapeDtypeStruct((b, _TOTAL), jnp.float32),
            jax.ShapeDtypeStruct((b,), jnp.int32),
        ],
    )(neg_s, posT, pcT, centers)
    return probs, choice


# speedup vs baseline: 2.2647x; 1.0074x over previous
import jax
import jax.numpy as jnp
from jax.experimental import pallas as pl

_N_SPHERES = 64
_TOTAL = 512
_ROWS = 2048


def _router_body(ns_ref, posT_ref, pcT_ref, c0_ref, probs_ref, choice_ref):
    neg_s = ns_ref[...]

    cx = c0_ref[:, 0:1]
    cy = c0_ref[:, 1:2]
    cz = c0_ref[:, 2:3]
    px = posT_ref[0:1, :]
    py = posT_ref[1:2, :]
    pz = posT_ref[2:3, :]
    dx = px - cx
    dy = py - cy
    dz = pz - cz
    d_sq = (dx * dx + dy * dy) + dz * dz
    v = d_sq / neg_s

    lpc = jnp.log(pcT_ref[...] + 1e-10)
    logits = jnp.concatenate(
        [v[g * _N_SPHERES:(g + 1) * _N_SPHERES, :] + lpc[g:g + 1, :]
         for g in range(8)], axis=0)

    srow = jax.lax.broadcasted_iota(jnp.int32, (_TOTAL, _ROWS), 0)
    m = jnp.max(logits, axis=0, keepdims=True)
    cand = jnp.where(logits == m, srow, _TOTAL)
    choiceT = jnp.min(cand, axis=0, keepdims=True)
    choice_ref[...] = choiceT.reshape(_ROWS)

    choice_col = choiceT.reshape(_ROWS, 1)
    lane = jax.lax.broadcasted_iota(jnp.int32, (_ROWS, _TOTAL), 1)
    probs_ref[...] = (lane == choice_col).astype(jnp.float32)


def kernel(pos_3d, temperature, parent_choice, hard, centers, log_radii):
    del hard, log_radii
    b = pos_3d.shape[0]
    neg_s = (-(2.0 * temperature**2 + 1e-8)).reshape(1, 1).astype(jnp.float32)
    posT = pos_3d.T
    pcT = parent_choice.T
    grid = (b // _ROWS,)
    probs, choice = pl.pallas_call(
        _router_body,
        grid=grid,
        in_specs=[
            pl.BlockSpec((1, 1), lambda i: (0, 0)),
            pl.BlockSpec((3, _ROWS), lambda i: (0, i)),
            pl.BlockSpec((8, _ROWS), lambda i: (0, i)),
            pl.BlockSpec((_TOTAL, 3), lambda i: (0, 0)),
        ],
        out_specs=[
            pl.BlockSpec((_ROWS, _TOTAL), lambda i: (i, 0)),
            pl.BlockSpec((_ROWS,), lambda i: (i,)),
        ],
        out_shape=[
            jax.ShapeDtypeStruct((b, _TOTAL), jnp.float32),
            jax.ShapeDtypeStruct((b,), jnp.int32),
        ],
    )(neg_s, posT, pcT, centers)
    return probs, choice
